# baseline (speedup 1.0000x reference)
import functools

import jax
import jax.numpy as jnp
from jax import lax
from jax.experimental import pallas as pl
from jax.experimental.pallas import tpu as pltpu
from jax.experimental.pallas import tpu_sc as plsc

BATCH = 4096
SEQ = 200
N_TOKENS = 10
EMBED_DIM = 64
LANES = 16

_SC_INFO = plsc.get_sparse_core_info()
NUM_WORKERS = _SC_INFO.num_cores * _SC_INFO.num_subcores
BT = BATCH // NUM_WORKERS
JT = EMBED_DIM // 8
NBUF = 2


@functools.partial(
    pl.kernel,
    out_type=jax.ShapeDtypeStruct((SEQ, JT, NUM_WORKERS, 8, BT), jnp.float32),
    mesh=plsc.VectorSubcoreMesh(core_axis_name="c", subcore_axis_name="s"),
    scratch_types=[
        pltpu.VMEM((SEQ * BT,), jnp.int32),
        pltpu.VMEM((NBUF, BT, EMBED_DIM), jnp.float32),
        pltpu.VMEM((NBUF, EMBED_DIM, BT), jnp.float32),
        pltpu.VMEM((N_TOKENS, EMBED_DIM), jnp.float32),
        pltpu.SemaphoreType.DMA,
        [pltpu.SemaphoreType.DMA] * NBUF,
        [pltpu.SemaphoreType.DMA] * NBUF,
    ],
    compiler_params=pltpu.CompilerParams(use_tc_tiling_on_sc=False,
                                         needs_layout_passes=False),
)
def _soft_embedding_sc(idx_hbm, table_hbm, learned_hbm, out_hbm,
                       idx_v, tbuf, obuf, learned_v, sem_i, sem_g, sem_w):
    wid = lax.axis_index("s") * _SC_INFO.num_cores + lax.axis_index("c")

    pltpu.sync_copy(learned_hbm, learned_v)
    pltpu.async_copy(idx_hbm.at[pl.ds(wid * SEQ * BT, SEQ * BT)],
                     idx_v, sem_i)
    rows_c = [lax.iota(jnp.int32, LANES) + (k * LANES)
              for k in range(BT // LANES)]
    pltpu.make_async_copy(idx_hbm.at[pl.ds(0, SEQ * BT)], idx_v, sem_i).wait()

    pltpu.async_copy(table_hbm.at[idx_v.at[pl.ds(0, BT)]],
                     tbuf.at[0], sem_g[0])

    def _write_waits(n):
        for jt in range(JT):
            pltpu.make_async_copy(
                obuf.at[n, pl.ds(jt * 8, 8)],
                out_hbm.at[0, jt, 0], sem_w[n]).wait()

    @pl.loop(0, SEQ, step=NBUF)
    def _(s):
        for n in range(NBUF):
            si = s + n
            nb = (n + 1) % NBUF

            @pl.when(si + 1 < SEQ)
            def _():
                pltpu.async_copy(
                    table_hbm.at[idx_v.at[pl.ds((si + 1) * BT, BT)]],
                    tbuf.at[nb], sem_g[nb])

            pltpu.make_async_copy(
                table_hbm.at[idx_v.at[pl.ds(0, BT)]],
                tbuf.at[n], sem_g[n]).wait()

            @pl.when(si >= NBUF)
            def _():
                _write_waits(n)

            @pl.when(si < N_TOKENS)
            def _():
                @pl.loop(0, EMBED_DIM, unroll=8)
                def _(j):
                    srow = jnp.full((LANES,), si, jnp.int32)
                    scol = jnp.full((LANES,), j, jnp.int32)
                    v = plsc.load_gather(learned_v, [srow, scol])
                    for k in range(BT // LANES):
                        obuf[n, j, pl.ds(k * LANES, LANES)] = v

            @pl.when(si >= N_TOKENS)
            def _():
                @pl.loop(0, EMBED_DIM, unroll=8)
                def _(j):
                    col = jnp.full((LANES,), j, jnp.int32)
                    for k in range(BT // LANES):
                        v = plsc.load_gather(tbuf.at[n], [rows_c[k], col])
                        obuf[n, j, pl.ds(k * LANES, LANES)] = v

            for jt in range(JT):
                pltpu.async_copy(
                    obuf.at[n, pl.ds(jt * 8, 8)],
                    out_hbm.at[si, jt, wid], sem_w[n])

    for n in range(NBUF):
        _write_waits(n)


def kernel(tokens, wte_weight, learned_embedding):
    tok32 = tokens.astype(jnp.int32)
    idx = tok32.T.reshape(SEQ, NUM_WORKERS, BT).transpose(1, 0, 2)
    idx = idx.reshape(NUM_WORKERS * SEQ * BT)
    out5 = _soft_embedding_sc(idx, wte_weight, learned_embedding)
    return out5.transpose(2, 4, 0, 1, 3).reshape(BATCH, SEQ, EMBED_DIM)


# speedup vs baseline: 1.4553x; 1.4553x over previous
import functools

import jax
import jax.numpy as jnp
from jax import lax
from jax.experimental import pallas as pl
from jax.experimental.pallas import tpu as pltpu
from jax.experimental.pallas import tpu_sc as plsc

BATCH = 4096
SEQ = 200
N_TOKENS = 10
EMBED_DIM = 64
NPAIR = SEQ // 2
CHUNK = 96
CHUNK1_START = SEQ - CHUNK
IDX_PER_ROW = 2 * CHUNK

_SC_INFO = plsc.get_sparse_core_info()
NUM_WORKERS = _SC_INFO.num_cores * _SC_INFO.num_subcores
ROWS_PER_WORKER = BATCH // NUM_WORKERS
R = 4
G = ROWS_PER_WORKER // R
NBUF = 2


@functools.partial(
    pl.kernel,
    out_type=jax.ShapeDtypeStruct((BATCH * SEQ, EMBED_DIM), jnp.float32),
    mesh=plsc.VectorSubcoreMesh(core_axis_name="c", subcore_axis_name="s"),
    scratch_types=[
        pltpu.VMEM((NBUF, R * IDX_PER_ROW), jnp.int32),
        pltpu.VMEM((NBUF, R * SEQ, EMBED_DIM), jnp.float32),
        [pltpu.SemaphoreType.DMA] * NBUF,
        [pltpu.SemaphoreType.DMA] * NBUF,
        [pltpu.SemaphoreType.DMA] * NBUF,
    ],
    compiler_params=pltpu.CompilerParams(use_tc_tiling_on_sc=False),
)
def _soft_embedding_sc(idx_hbm, table_hbm, learned_hbm, out_hbm,
                       idx_v, buf_v, sem_idx, sem_g, sem_out):
    wid = lax.axis_index("s") * _SC_INFO.num_cores + lax.axis_index("c")
    base = wid * ROWS_PER_WORKER

    for n in range(NBUF):
        for r in range(R):
            pltpu.sync_copy(learned_hbm,
                            buf_v.at[n, pl.ds(r * SEQ, N_TOKENS)])

    pltpu.async_copy(idx_hbm.at[pl.ds(base * IDX_PER_ROW, R * IDX_PER_ROW)],
                     idx_v.at[0], sem_idx[0])

    @pl.loop(0, G, step=NBUF)
    def _(g):
        for n in range(NBUF):
            gi = g + n
            nb = (n + 1) % NBUF

            @pl.when(gi + 1 < G)
            def _():
                pltpu.async_copy(
                    idx_hbm.at[pl.ds((base + (gi + 1) * R) * IDX_PER_ROW,
                                     R * IDX_PER_ROW)],
                    idx_v.at[nb], sem_idx[nb])

            pltpu.make_async_copy(
                idx_hbm.at[pl.ds(0, R * IDX_PER_ROW)],
                idx_v.at[n], sem_idx[n]).wait()

            @pl.when(gi >= NBUF)
            def _():
                pltpu.make_async_copy(
                    buf_v.at[n], out_hbm.at[pl.ds(0, R * SEQ)],
                    sem_out[n]).wait()

            for r in range(R):
                pltpu.async_copy(
                    table_hbm.at[idx_v.at[n, pl.ds(r * IDX_PER_ROW, CHUNK)]],
                    buf_v.at[n, pl.ds(r * SEQ + N_TOKENS, CHUNK)], sem_g[n])
                pltpu.async_copy(
                    table_hbm.at[idx_v.at[n, pl.ds(r * IDX_PER_ROW + CHUNK,
                                                   CHUNK)]],
                    buf_v.at[n, pl.ds(r * SEQ + CHUNK1_START, CHUNK)],
                    sem_g[n])
            for r in range(R):
                pltpu.make_async_copy(
                    table_hbm.at[idx_v.at[n, pl.ds(r * IDX_PER_ROW, CHUNK)]],
                    buf_v.at[n, pl.ds(r * SEQ + N_TOKENS, CHUNK)],
                    sem_g[n]).wait()
                pltpu.make_async_copy(
                    table_hbm.at[idx_v.at[n, pl.ds(r * IDX_PER_ROW + CHUNK,
                                                   CHUNK)]],
                    buf_v.at[n, pl.ds(r * SEQ + CHUNK1_START, CHUNK)],
                    sem_g[n]).wait()

            pltpu.async_copy(
                buf_v.at[n],
                out_hbm.at[pl.ds((base + gi * R) * SEQ, R * SEQ)],
                sem_out[n])

    for n in range(NBUF):
        pltpu.make_async_copy(
            buf_v.at[n], out_hbm.at[pl.ds(0, R * SEQ)], sem_out[n]).wait()


def kernel(tokens, wte_weight, learned_embedding):
    tok32 = tokens.astype(jnp.int32)
    idx = jnp.stack(
        [tok32[:, N_TOKENS:N_TOKENS + CHUNK],
         tok32[:, CHUNK1_START:SEQ]], axis=1)
    idx = idx.reshape(BATCH * IDX_PER_ROW)
    out = _soft_embedding_sc(idx, wte_weight, learned_embedding)
    return out.reshape(BATCH, SEQ, EMBED_DIM)
